# baseline (speedup 1.0000x reference)
import jax
import jax.numpy as jnp
from jax import lax
from jax.experimental import pallas as pl
from jax.experimental.pallas import tpu as pltpu

SEQ = 81
VOCAB = 10
HID = 512
NB = 64
NBUF = 4


def _compute(x, tok, pos):
    shape = (NB, SEQ, HID)
    xb = jnp.broadcast_to(x[:, :, None], shape)

    def tv(v):
        return jnp.broadcast_to(tok[v, :][None, None, :], shape)

    m0 = (xb & 1) != 0
    m1 = (xb & 2) != 0
    m2 = (xb & 4) != 0
    m3 = (xb & 8) != 0
    t01 = jnp.where(m0, tv(1), tv(0))
    t23 = jnp.where(m0, tv(3), tv(2))
    t45 = jnp.where(m0, tv(5), tv(4))
    t67 = jnp.where(m0, tv(7), tv(6))
    t89 = jnp.where(m0, tv(9), tv(8))
    t03 = jnp.where(m1, t23, t01)
    t47 = jnp.where(m1, t67, t45)
    t07 = jnp.where(m2, t47, t03)
    tok_sel = jnp.where(m3, t89, t07)
    return tok_sel + jnp.broadcast_to(pos[None, :, :], shape)


def _body(x_hbm, tok_ref, pos_ref, out_hbm, x_all, b0, b1, b2, b3,
          in_sem, out_sems):
    nblk = x_hbm.shape[0] // NB
    nround = nblk // NBUF
    slots = (b0, b1, b2, b3)
    pltpu.make_async_copy(x_hbm, x_all, in_sem).start()
    pltpu.make_async_copy(x_hbm, x_all, in_sem).wait()
    tok = tok_ref[...]
    pos = pos_ref[...]

    def round_(r, _):
        for s in range(NBUF):
            i = r * NBUF + s
            buf = slots[s]

            @pl.when(r > 0)
            def _wait_prev():
                pltpu.make_async_copy(
                    buf,
                    out_hbm.at[pl.ds((i - NBUF) * NB, NB)],
                    out_sems.at[s],
                ).wait()

            buf[...] = _compute(x_all[pl.ds(i * NB, NB), :], tok, pos)
            pltpu.make_async_copy(
                buf,
                out_hbm.at[pl.ds(i * NB, NB)],
                out_sems.at[s],
            ).start()
        return 0

    lax.fori_loop(0, nround, round_, 0)

    for s in range(NBUF):
        i = (nround - 1) * NBUF + s
        pltpu.make_async_copy(
            slots[s],
            out_hbm.at[pl.ds(i * NB, NB)],
            out_sems.at[s],
        ).wait()


def kernel(x, token_emb, pos_emb):
    B = x.shape[0]
    out = pl.pallas_call(
        _body,
        in_specs=[
            pl.BlockSpec(memory_space=pl.ANY),
            pl.BlockSpec(memory_space=pltpu.MemorySpace.VMEM),
            pl.BlockSpec(memory_space=pltpu.MemorySpace.VMEM),
        ],
        out_specs=pl.BlockSpec(memory_space=pl.ANY),
        out_shape=jax.ShapeDtypeStruct((B, SEQ, HID), jnp.float32),
        scratch_shapes=[
            pltpu.VMEM((B, SEQ), jnp.int32),
            pltpu.VMEM((NB, SEQ, HID), jnp.float32),
            pltpu.VMEM((NB, SEQ, HID), jnp.float32),
            pltpu.VMEM((NB, SEQ, HID), jnp.float32),
            pltpu.VMEM((NB, SEQ, HID), jnp.float32),
            pltpu.SemaphoreType.DMA,
            pltpu.SemaphoreType.DMA((NBUF,)),
        ],
    )(x, token_emb, pos_emb)
    return out
